# Initial kernel scaffold; baseline (speedup 1.0000x reference)
import functools

import jax
import jax.numpy as jnp
from jax import lax
from jax.experimental import pallas as pl
from jax.experimental.pallas import tpu as pltpu
from jax.experimental.pallas import tpu_sc as plsc

_NC = 2
_NS = 16
_NW = _NC * _NS

_IDX_W = 128
_K = 8
_CHUNK = _K * _IDX_W


def _sc_gather_body(idx_hbm, table_hbm, out_hbm, idx_v, rows_v, sem):
  n_total = out_hbm.shape[0]
  b_per_w = n_total // _NW
  n_chunks = b_per_w // _CHUNK
  idx_rows_per_w = b_per_w // _IDX_W

  wid = lax.axis_index("s") * _NC + lax.axis_index("c")
  row0 = wid * idx_rows_per_w
  base0 = wid * b_per_w

  def chunk(c, carry):
    pltpu.sync_copy(idx_hbm.at[pl.ds(row0 + c * _K, _K)], idx_v)
    cps = []
    for j in range(_K):
      cps.append(
          pltpu.async_copy(
              table_hbm.at[idx_v.at[j]],
              rows_v.at[pl.ds(j * _IDX_W, _IDX_W)],
              sem,
          )
      )
    for cp in cps:
      cp.wait()
    pltpu.sync_copy(rows_v, out_hbm.at[pl.ds(base0 + c * _CHUNK, _CHUNK)])
    return carry

  lax.fori_loop(0, n_chunks, chunk, 0)


def _sc_gather(flat_idx, emb_table):
  n_total = flat_idx.shape[0]
  d = emb_table.shape[1]
  idx2d = flat_idx.reshape(n_total // _IDX_W, _IDX_W)
  mesh = plsc.VectorSubcoreMesh(core_axis_name="c", subcore_axis_name="s")
  f = functools.partial(
      pl.kernel,
      out_type=jax.ShapeDtypeStruct((n_total, d), jnp.float32),
      mesh=mesh,
      scratch_types=[
          pltpu.VMEM((_K, _IDX_W), jnp.int32),
          pltpu.VMEM((_CHUNK, d), jnp.float32),
          pltpu.SemaphoreType.DMA,
      ],
  )(_sc_gather_body)
  return f(idx2d, emb_table)


def _masklen_body(x_ref, mask_ref, len_ref):
  x = x_ref[...]
  m = x != 0
  mask_ref[...] = m
  len_ref[...] = jnp.sum(m.astype(jnp.int32), axis=1)


def _tc_masklen(inputs):
  b, h = inputs.shape
  rb = 512
  grid = (b // rb,)
  return pl.pallas_call(
      _masklen_body,
      grid=grid,
      in_specs=[pl.BlockSpec((rb, h), lambda i: (i, 0))],
      out_specs=[
          pl.BlockSpec((rb, h), lambda i: (i, 0)),
          pl.BlockSpec((rb,), lambda i: (i,)),
      ],
      out_shape=[
          jax.ShapeDtypeStruct((b, h), jnp.bool_),
          jax.ShapeDtypeStruct((b,), jnp.int32),
      ],
  )(inputs)


@jax.jit
def kernel(inputs, emb_table):
  b, h = inputs.shape
  d = emb_table.shape[1]
  emb_flat = _sc_gather(inputs.reshape(-1), emb_table)
  masks, lengths = _tc_masklen(inputs)
  return emb_flat.reshape(b, h, d), lengths, masks


# speedup vs baseline: 4.1379x; 4.1379x over previous
import functools

import jax
import jax.numpy as jnp
from jax import lax
from jax.experimental import pallas as pl
from jax.experimental.pallas import tpu as pltpu
from jax.experimental.pallas import tpu_sc as plsc

_NC = 2
_NS = 16
_NW = _NC * _NS

_IDX_W = 128
_K = 8
_CHUNK = _K * _IDX_W


def _sc_gather_body(idx_hbm, table_hbm, out_hbm, idx_v, rows_v, sem):
  n_total = out_hbm.shape[0]
  b_per_w = n_total // _NW
  n_chunks = b_per_w // _CHUNK
  idx_rows_per_w = b_per_w // _IDX_W

  wid = lax.axis_index("s") * _NC + lax.axis_index("c")
  row0 = wid * idx_rows_per_w
  base0 = wid * b_per_w

  def chunk(c, carry):
    pltpu.sync_copy(idx_hbm.at[pl.ds(row0 + c * _K, _K)], idx_v)
    cps = []
    for j in range(_K):
      cps.append(
          pltpu.async_copy(
              table_hbm.at[idx_v.at[j]],
              rows_v.at[pl.ds(j * _IDX_W, _IDX_W)],
              sem,
          )
      )
    for cp in cps:
      cp.wait()
    pltpu.sync_copy(rows_v, out_hbm.at[pl.ds(base0 + c * _CHUNK, _CHUNK)])
    return carry

  lax.fori_loop(0, n_chunks, chunk, 0)


def _sc_gather(flat_idx, emb_table):
  n_total = flat_idx.shape[0]
  d = emb_table.shape[1]
  idx2d = flat_idx.reshape(n_total // _IDX_W, _IDX_W)
  mesh = plsc.VectorSubcoreMesh(core_axis_name="c", subcore_axis_name="s")
  f = functools.partial(
      pl.kernel,
      out_type=jax.ShapeDtypeStruct((n_total, d), jnp.float32),
      mesh=mesh,
      scratch_types=[
          pltpu.VMEM((_K, _IDX_W), jnp.int32),
          pltpu.VMEM((_CHUNK, d), jnp.float32),
          pltpu.SemaphoreType.DMA,
      ],
      compiler_params=pltpu.CompilerParams(use_tc_tiling_on_sc=False),
  )(_sc_gather_body)
  return f(idx2d, emb_table)


def _masklen_body(x_ref, mask_ref, len_ref):
  x = x_ref[...]
  m = x != 0
  mask_ref[...] = m
  len_ref[...] = jnp.sum(m.astype(jnp.int32), axis=1)


def _tc_masklen(inputs):
  b, h = inputs.shape
  rb = 512
  grid = (b // rb,)
  return pl.pallas_call(
      _masklen_body,
      grid=grid,
      in_specs=[pl.BlockSpec((rb, h), lambda i: (i, 0))],
      out_specs=[
          pl.BlockSpec((rb, h), lambda i: (i, 0)),
          pl.BlockSpec((rb,), lambda i: (i,)),
      ],
      out_shape=[
          jax.ShapeDtypeStruct((b, h), jnp.bool_),
          jax.ShapeDtypeStruct((b,), jnp.int32),
      ],
  )(inputs)


@jax.jit
def kernel(inputs, emb_table):
  b, h = inputs.shape
  d = emb_table.shape[1]
  emb_flat = _sc_gather(inputs.reshape(-1), emb_table)
  masks, lengths = _tc_masklen(inputs)
  return emb_flat.reshape(b, h, d), lengths, masks


# speedup vs baseline: 4.1960x; 1.0140x over previous
import functools

import jax
import jax.numpy as jnp
from jax import lax
from jax.experimental import pallas as pl
from jax.experimental.pallas import tpu as pltpu
from jax.experimental.pallas import tpu_sc as plsc

_NC = 2
_NS = 16
_NW = _NC * _NS

_IDX_W = 128
_K = 5
_CHUNK = _K * _IDX_W


def _sc_gather_body(idx_hbm, table_hbm, out_hbm, idx_v, rows_v, sem_g,
                    sem_o0, sem_o1):
  n_total = out_hbm.shape[0]
  d = out_hbm.shape[1]
  b_per_w = n_total // _NW
  n_chunks = b_per_w // _CHUNK
  idx_rows_per_w = b_per_w // _IDX_W

  wid = lax.axis_index("s") * _NC + lax.axis_index("c")
  row0 = wid * idx_rows_per_w
  base0 = wid * b_per_w

  sem_o = (sem_o0, sem_o1)

  def pair(i, carry):
    for b in range(2):
      c = 2 * i + b

      @pl.when(i > 0)
      def _():
        pltpu.make_async_copy(
            out_hbm.at[pl.ds(0, _CHUNK)], rows_v.at[b], sem_o[b]
        ).wait()

      pltpu.sync_copy(idx_hbm.at[pl.ds(row0 + c * _K, _K)], idx_v.at[b])
      cps = []
      for j in range(_K):
        cps.append(
            pltpu.async_copy(
                table_hbm.at[idx_v.at[b].at[j]],
                rows_v.at[b].at[pl.ds(j * _IDX_W, _IDX_W)],
                sem_g,
            )
        )
      for cp in cps:
        cp.wait()
      pltpu.async_copy(
          rows_v.at[b], out_hbm.at[pl.ds(base0 + c * _CHUNK, _CHUNK)],
          sem_o[b],
      )
    return carry

  lax.fori_loop(0, n_chunks // 2, pair, 0)

  for b in range(2):
    pltpu.make_async_copy(
        out_hbm.at[pl.ds(0, _CHUNK)], rows_v.at[b], sem_o[b]
    ).wait()


def _sc_gather(flat_idx, emb_table):
  n_total = flat_idx.shape[0]
  d = emb_table.shape[1]
  idx2d = flat_idx.reshape(n_total // _IDX_W, _IDX_W)
  mesh = plsc.VectorSubcoreMesh(core_axis_name="c", subcore_axis_name="s")
  f = functools.partial(
      pl.kernel,
      out_type=jax.ShapeDtypeStruct((n_total, d), jnp.float32),
      mesh=mesh,
      scratch_types=[
          pltpu.VMEM((2, _K, _IDX_W), jnp.int32),
          pltpu.VMEM((2, _CHUNK, d), jnp.float32),
          pltpu.SemaphoreType.DMA,
          pltpu.SemaphoreType.DMA,
          pltpu.SemaphoreType.DMA,
      ],
      compiler_params=pltpu.CompilerParams(use_tc_tiling_on_sc=False),
  )(_sc_gather_body)
  return f(idx2d, emb_table)


def _masklen_body(x_ref, mask_ref, len_ref):
  x = x_ref[...]
  m = x != 0
  mask_ref[...] = m
  len_ref[...] = jnp.sum(m.astype(jnp.int32), axis=1)


def _tc_masklen(inputs):
  b, h = inputs.shape
  rb = 512
  grid = (b // rb,)
  return pl.pallas_call(
      _masklen_body,
      grid=grid,
      in_specs=[pl.BlockSpec((rb, h), lambda i: (i, 0))],
      out_specs=[
          pl.BlockSpec((rb, h), lambda i: (i, 0)),
          pl.BlockSpec((rb,), lambda i: (i,)),
      ],
      out_shape=[
          jax.ShapeDtypeStruct((b, h), jnp.bool_),
          jax.ShapeDtypeStruct((b,), jnp.int32),
      ],
  )(inputs)


@jax.jit
def kernel(inputs, emb_table):
  b, h = inputs.shape
  d = emb_table.shape[1]
  emb_flat = _sc_gather(inputs.reshape(-1), emb_table)
  masks, lengths = _tc_masklen(inputs)
  return emb_flat.reshape(b, h, d), lengths, masks
